# Initial kernel scaffold; baseline (speedup 1.0000x reference)
import jax
import jax.numpy as jnp
from jax.experimental import pallas as pl


def kernel(x, edge_index, batch, W1l, W1r, W2l, W2r, Wfc):
    raise NotImplementedError("write your pallas kernel here")


# speedup vs baseline: 20.9816x; 20.9816x over previous
import functools

import jax
import jax.numpy as jnp
from jax import lax
from jax.experimental import pallas as pl
from jax.experimental.pallas import tpu as pltpu
from jax.experimental.pallas import tpu_sc as plsc

NC = 2
NS = 16
NW = NC * NS
LANE = 128
K_CH = 16
DP = 16


def _pre_body(one_col, x_ref, wl_ref, wr_ref, p_ref, q_ref):
    xb = x_ref[...]
    p = jnp.dot(xb, wl_ref[...], preferred_element_type=jnp.float32)
    col = lax.broadcasted_iota(jnp.int32, p.shape, 1)
    p_ref[...] = jnp.where(col == one_col, 1.0, p)
    q_ref[...] = jnp.dot(xb, wr_ref[...], preferred_element_type=jnp.float32)


def _mid_body(one_col, acc_ref, q_ref, wl_ref, wr_ref, p_ref, q2_ref):
    a = acc_ref[0] + acc_ref[1]
    deg = jnp.maximum(a[:, one_col:one_col + 1], 1.0)
    h = jnp.maximum(a / deg + q_ref[...], 0.0)
    p = jnp.dot(h, wl_ref[...], preferred_element_type=jnp.float32)
    col = lax.broadcasted_iota(jnp.int32, p.shape, 1)
    p_ref[...] = jnp.where(col == one_col, 1.0, p)
    q2_ref[...] = jnp.dot(h, wr_ref[...], preferred_element_type=jnp.float32)


def _post_body(one_col, n_graphs, acc_ref, q_ref, batch_ref, wfc_ref, out_ref):
    a = acc_ref[0] + acc_ref[1]
    deg = jnp.maximum(a[:, one_col:one_col + 1], 1.0)
    h2 = a / deg + q_ref[...]
    rows = lax.broadcasted_iota(jnp.int32, (n_graphs, h2.shape[0]), 0)
    oh = (rows == batch_ref[...]).astype(jnp.float32)
    g = jnp.dot(oh, h2, preferred_element_type=jnp.float32)
    cnt = jnp.maximum(g[:, one_col:one_col + 1], 1.0)
    out_ref[...] = jax.nn.sigmoid(
        jnp.dot(g / cnt, wfc_ref[...], preferred_element_type=jnp.float32))


def _sc_scatter(n_nodes, n_acc, rows_per_tile,
                msg_hbm, src_hbm, dst_hbm, zeros_hbm, out_hbm,
                src_v, dst_v, rows_v, table_sh, acc_sh, sem):
    c = lax.axis_index("c")
    s = lax.axis_index("s")
    wid = s * NC + c
    tr = -(-(n_nodes // NS) // 8) * 8
    ar = -(-(n_acc // NS) // 8) * 8
    t_off = pl.multiple_of(jnp.minimum(s * tr, n_nodes - tr), 8)
    a_off = pl.multiple_of(jnp.minimum(s * ar, n_acc - ar), 8)
    pltpu.sync_copy(msg_hbm.at[pl.ds(t_off, tr)], table_sh.at[pl.ds(t_off, tr)])
    pltpu.sync_copy(zeros_hbm.at[pl.ds(a_off, ar)], acc_sh.at[pl.ds(a_off, ar)])
    plsc.subcore_barrier()

    base = wid * rows_per_tile

    def chunk(i, carry):
        row0 = base + i * K_CH
        pltpu.sync_copy(src_hbm.at[pl.ds(row0, K_CH)], src_v)
        pltpu.sync_copy(dst_hbm.at[pl.ds(row0, K_CH)], dst_v)
        for j in range(K_CH):
            sl = pl.ds(j * LANE, LANE)
            pltpu.async_copy(table_sh.at[src_v.at[j]], rows_v.at[sl], sem).wait()
            pltpu.sync_copy(rows_v.at[sl], acc_sh.at[dst_v.at[j]], add=True)
        return carry

    lax.fori_loop(0, rows_per_tile // K_CH, chunk, 0)
    plsc.subcore_barrier()
    pltpu.sync_copy(acc_sh.at[pl.ds(t_off, tr)],
                    out_hbm.at[c, pl.ds(t_off, tr)])


def _make_sc_pass(n_nodes, n_acc, rows_per_tile):
    mesh = plsc.VectorSubcoreMesh(core_axis_name="c", subcore_axis_name="s")
    return pl.kernel(
        functools.partial(_sc_scatter, n_nodes, n_acc, rows_per_tile),
        out_type=jax.ShapeDtypeStruct((NC, n_nodes, DP), jnp.float32),
        mesh=mesh,
        scratch_types=[
            pltpu.VMEM((K_CH, LANE), jnp.int32),
            pltpu.VMEM((K_CH, LANE), jnp.int32),
            pltpu.VMEM((K_CH * LANE, DP), jnp.float32),
            pltpu.VMEM_SHARED((n_nodes, DP), jnp.float32),
            pltpu.VMEM_SHARED((n_acc, DP), jnp.float32),
            pltpu.SemaphoreType.DMA,
        ],
        compiler_params=pltpu.CompilerParams(use_tc_tiling_on_sc=False),
    )


def kernel(x, edge_index, batch, W1l, W1r, W2l, W2r, Wfc):
    n, f = x.shape
    e = edge_index.shape[1]
    d = W1l.shape[1]
    g = 128

    wl1 = jnp.zeros((f, DP), jnp.float32).at[:, :d].set(W1l)
    wr1 = jnp.zeros((f, DP), jnp.float32).at[:, :d].set(W1r)
    wl2 = jnp.zeros((DP, DP), jnp.float32).at[:d, :d].set(W2l)
    wr2 = jnp.zeros((DP, DP), jnp.float32).at[:d, :d].set(W2r)
    wfc = jnp.zeros((DP, 8), jnp.float32).at[:d, :1].set(Wfc)

    rows = -(-e // LANE)
    rows_pad = -(-rows // (NW * K_CH)) * (NW * K_CH)
    pad = rows_pad * LANE - e
    src = jnp.concatenate([edge_index[0], jnp.zeros((pad,), jnp.int32)])
    dst = jnp.concatenate([edge_index[1], jnp.full((pad,), n, jnp.int32)])
    src2 = src.reshape(rows_pad, LANE)
    dst2 = dst.reshape(rows_pad, LANE)
    n_acc = n + NS
    zeros = jnp.zeros((n_acc, DP), jnp.float32)
    rows_per_tile = rows_pad // NW

    pre = pl.pallas_call(
        functools.partial(_pre_body, d),
        out_shape=[jax.ShapeDtypeStruct((n, DP), jnp.float32)] * 2,
    )
    p1, q1 = pre(x, wl1, wr1)

    sc_pass = _make_sc_pass(n, n_acc, rows_per_tile)
    acc1 = sc_pass(p1, src2, dst2, zeros)

    mid = pl.pallas_call(
        functools.partial(_mid_body, d),
        out_shape=[jax.ShapeDtypeStruct((n, DP), jnp.float32)] * 2,
    )
    p2, q2 = mid(acc1, q1, wl2, wr2)

    acc2 = sc_pass(p2, src2, dst2, zeros)

    post = pl.pallas_call(
        functools.partial(_post_body, d, g),
        out_shape=jax.ShapeDtypeStruct((g, 8), jnp.float32),
    )
    y = post(acc2, q2, batch.reshape(1, n), wfc)
    return y[:, :1]


# speedup vs baseline: 26.4750x; 1.2618x over previous
import functools

import jax
import jax.numpy as jnp
from jax import lax
from jax.experimental import pallas as pl
from jax.experimental.pallas import tpu as pltpu
from jax.experimental.pallas import tpu_sc as plsc

NC = 2
NS = 16
NW = NC * NS
LANE = 128
K_CH = 16
DP = 16
PK = LANE // DP


def _pre_body(one_col, x_ref, wl_ref, wr_ref, p_ref, q_ref):
    xb = x_ref[...]
    p = jnp.dot(xb, wl_ref[...], preferred_element_type=jnp.float32)
    col = lax.broadcasted_iota(jnp.int32, p.shape, 1)
    p_ref[...] = jnp.where(col % DP == one_col, 1.0, p)
    q_ref[...] = jnp.dot(xb, wr_ref[...], preferred_element_type=jnp.float32)


def _mid_body(acc_ref, q_ref, s_ref, wl_ref, wr_ref, p_ref, q2_ref):
    a = acc_ref[0] + acc_ref[1]
    degb = jnp.maximum(jnp.dot(a, s_ref[...],
                               preferred_element_type=jnp.float32), 1.0)
    h = jnp.maximum(a / degb + q_ref[...], 0.0)
    p_ref[...] = jnp.dot(h, wl_ref[...], preferred_element_type=jnp.float32)
    q2_ref[...] = jnp.dot(h, wr_ref[...], preferred_element_type=jnp.float32)


def _post_body(acc_ref, q_ref, s_ref, wz_ref, b_ref, out_ref):
    a = acc_ref[0] + acc_ref[1]
    degb = jnp.maximum(jnp.dot(a, s_ref[...],
                               preferred_element_type=jnp.float32), 1.0)
    h2 = a / degb + q_ref[...]
    z = jnp.dot(h2, wz_ref[...], preferred_element_type=jnp.float32)
    n_graphs = out_ref.shape[0]
    gid = lax.broadcasted_iota(jnp.int32, (n_graphs, b_ref.shape[1]), 0)
    ys = jnp.zeros((n_graphs, 1), jnp.float32)
    cn = jnp.zeros((n_graphs, 1), jnp.float32)
    for j in range(PK):
        oh = (gid == b_ref[j:j + 1, :]).astype(jnp.float32)
        zj = jnp.dot(oh, z, preferred_element_type=jnp.float32)
        ys = ys + zj[:, j * DP:j * DP + 1]
        cn = cn + zj[:, j * DP + 8:j * DP + 9]
    out_ref[...] = jax.nn.sigmoid(ys / jnp.maximum(cn, 1.0))


def _sc_scatter(n_nodes, n_acc, n_rows, rows_per_tile,
                msg_hbm, edge_hbm, zeros_hbm, out_hbm,
                src_v, dst_v, rows_v, table_sh, acc_sh, sem):
    c = lax.axis_index("c")
    s = lax.axis_index("s")
    wid = s * NC + c
    tr = -(-(n_nodes // NS) // 8) * 8
    ar = -(-(n_acc // NS) // 8) * 8
    t_off = pl.multiple_of(jnp.minimum(s * tr, n_nodes - tr), 8)
    a_off = pl.multiple_of(jnp.minimum(s * ar, n_acc - ar), 8)
    pltpu.sync_copy(msg_hbm.at[pl.ds(t_off, tr)], table_sh.at[pl.ds(t_off, tr)])
    pltpu.sync_copy(zeros_hbm.at[pl.ds(a_off, ar)], acc_sh.at[pl.ds(a_off, ar)])
    plsc.subcore_barrier()

    base = wid * rows_per_tile
    dummy = n_nodes + lax.iota(jnp.int32, DP)

    def chunk(i, carry):
        row0 = base + i * K_CH
        row0c = jnp.minimum(row0, n_rows - K_CH)
        dup = row0 - row0c
        pltpu.sync_copy(edge_hbm.at[0, pl.ds(row0c, K_CH)], src_v)
        pltpu.sync_copy(edge_hbm.at[1, pl.ds(row0c, K_CH)], dst_v)
        for j in range(K_CH):
            @pl.when(j < dup)
            def _():
                for k in range(LANE // DP):
                    dst_v[j, pl.ds(k * DP, DP)] = dummy
            sl = pl.ds(j * LANE, LANE)
            pltpu.async_copy(table_sh.at[src_v.at[j]], rows_v.at[sl], sem).wait()
            pltpu.sync_copy(rows_v.at[sl], acc_sh.at[dst_v.at[j]], add=True)
        return carry

    lax.fori_loop(0, rows_per_tile // K_CH, chunk, 0)
    plsc.subcore_barrier()
    pltpu.sync_copy(acc_sh.at[pl.ds(t_off, tr)],
                    out_hbm.at[c, pl.ds(t_off, tr)])


def _make_sc_pass(n_nodes, n_acc, n_rows, rows_per_tile):
    mesh = plsc.VectorSubcoreMesh(core_axis_name="c", subcore_axis_name="s")
    return pl.kernel(
        functools.partial(_sc_scatter, n_nodes, n_acc, n_rows, rows_per_tile),
        out_type=jax.ShapeDtypeStruct((NC, n_nodes, DP), jnp.float32),
        mesh=mesh,
        scratch_types=[
            pltpu.VMEM((K_CH, LANE), jnp.int32),
            pltpu.VMEM((K_CH, LANE), jnp.int32),
            pltpu.VMEM((K_CH * LANE, DP), jnp.float32),
            pltpu.VMEM_SHARED((n_nodes, DP), jnp.float32),
            pltpu.VMEM_SHARED((n_acc, DP), jnp.float32),
            pltpu.SemaphoreType.DMA,
        ],
        compiler_params=pltpu.CompilerParams(use_tc_tiling_on_sc=False),
    )


def _block_diag8(w16):
    return jnp.einsum("ab,kc->akbc", jnp.eye(PK, dtype=jnp.float32),
                      w16).reshape(LANE, LANE)


def kernel(x, edge_index, batch, W1l, W1r, W2l, W2r, Wfc):
    n, f = x.shape
    e = edge_index.shape[1]
    d = W1l.shape[1]
    g = 128
    npk = n // PK
    n_rows = e // LANE
    rows_per_tile = -(-n_rows // (NW * K_CH)) * K_CH
    n_acc = n + DP

    w1l_p = jnp.zeros((f, DP), jnp.float32).at[:, :d].set(W1l)
    w1r_p = jnp.zeros((f, DP), jnp.float32).at[:, :d].set(W1r)
    eye8 = jnp.eye(PK, dtype=jnp.float32)
    w1l_big = jnp.einsum("ab,kc->akbc", eye8, w1l_p).reshape(PK * f, LANE)
    w1r_big = jnp.einsum("ab,kc->akbc", eye8, w1r_p).reshape(PK * f, LANE)
    w2l_p = jnp.zeros((DP, DP), jnp.float32).at[:d, :d].set(W2l).at[d, d].set(1.0)
    w2r_p = jnp.zeros((DP, DP), jnp.float32).at[:d, :d].set(W2r)
    w2l_big = _block_diag8(w2l_p)
    w2r_big = _block_diag8(w2r_p)
    sel = jnp.zeros((DP, DP), jnp.float32).at[d, :].set(1.0)
    s_big = _block_diag8(sel)
    wz_p = jnp.zeros((DP, DP), jnp.float32).at[:d, 0].set(Wfc[:, 0]).at[d, 8].set(1.0)
    wz_big = _block_diag8(wz_p)

    x_pk = x.reshape(npk, PK * f)
    edge3 = edge_index.reshape(2, n_rows, LANE)
    batch_r8 = batch.reshape(npk, PK).T
    zeros = jnp.zeros((n_acc, DP), jnp.float32)

    pre = pl.pallas_call(
        functools.partial(_pre_body, d),
        out_shape=[jax.ShapeDtypeStruct((npk, LANE), jnp.float32)] * 2,
    )
    p1, q1 = pre(x_pk, w1l_big, w1r_big)

    sc_pass = _make_sc_pass(n, n_acc, n_rows, rows_per_tile)
    acc1 = sc_pass(p1.reshape(n, DP), edge3, zeros)

    mid = pl.pallas_call(
        _mid_body,
        out_shape=[jax.ShapeDtypeStruct((npk, LANE), jnp.float32)] * 2,
    )
    p2, q2 = mid(acc1.reshape(NC, npk, LANE), q1, s_big, w2l_big, w2r_big)

    acc2 = sc_pass(p2.reshape(n, DP), edge3, zeros)

    post = pl.pallas_call(
        _post_body,
        out_shape=jax.ShapeDtypeStruct((g, 1), jnp.float32),
    )
    return post(acc2.reshape(NC, npk, LANE), q2, s_big, wz_big, batch_r8)


# speedup vs baseline: 32.1216x; 1.2133x over previous
import functools

import jax
import jax.numpy as jnp
from jax import lax
from jax.experimental import pallas as pl
from jax.experimental.pallas import tpu as pltpu
from jax.experimental.pallas import tpu_sc as plsc

NC = 2
NS = 16
NW = NC * NS
LANE = 128
K_CH = 16
DP = 16
PK = LANE // DP


def _pre_body(one_col, x_ref, wl_ref, wr_ref, p_ref, q_ref):
    xb = x_ref[...]
    p = jnp.dot(xb, wl_ref[...], preferred_element_type=jnp.float32)
    col = lax.broadcasted_iota(jnp.int32, p.shape, 1)
    p_ref[...] = jnp.where(col % DP == one_col, 1.0, p)
    q_ref[...] = jnp.dot(xb, wr_ref[...], preferred_element_type=jnp.float32)


def _mid_body(acc_ref, q_ref, s_ref, wl_ref, wr_ref, p_ref, q2_ref):
    a = acc_ref[0] + acc_ref[1]
    degb = jnp.maximum(jnp.dot(a, s_ref[...],
                               preferred_element_type=jnp.float32), 1.0)
    h = jnp.maximum(a / degb + q_ref[...], 0.0)
    p_ref[...] = jnp.dot(h, wl_ref[...], preferred_element_type=jnp.float32)
    q2_ref[...] = jnp.dot(h, wr_ref[...], preferred_element_type=jnp.float32)


def _post_body(acc_ref, q_ref, s_ref, wz_ref, b_ref, out_ref):
    a = acc_ref[0] + acc_ref[1]
    degb = jnp.maximum(jnp.dot(a, s_ref[...],
                               preferred_element_type=jnp.float32), 1.0)
    h2 = a / degb + q_ref[...]
    z = jnp.dot(h2, wz_ref[...], preferred_element_type=jnp.float32)
    n_graphs = out_ref.shape[0]
    gid = lax.broadcasted_iota(jnp.int32, (n_graphs, b_ref.shape[1]), 0)
    ys = jnp.zeros((n_graphs, 1), jnp.float32)
    cn = jnp.zeros((n_graphs, 1), jnp.float32)
    for j in range(PK):
        oh = (gid == b_ref[j:j + 1, :]).astype(jnp.float32)
        zj = jnp.dot(oh, z, preferred_element_type=jnp.float32)
        ys = ys + zj[:, j * DP:j * DP + 1]
        cn = cn + zj[:, j * DP + 8:j * DP + 9]
    out_ref[...] = jax.nn.sigmoid(ys / jnp.maximum(cn, 1.0))


def _sc_scatter(n_nodes, n_acc, n_rows, rows_per_tile,
                msg_hbm, edge_hbm, zeros_hbm, out_hbm,
                src_v, dst_v, rows_v, table_sh, acc_sh, sem):
    c = lax.axis_index("c")
    s = lax.axis_index("s")
    wid = s * NC + c
    tr = -(-(n_nodes // NS) // 8) * 8
    ar = -(-(n_acc // NS) // 8) * 8
    t_off = pl.multiple_of(jnp.minimum(s * tr, n_nodes - tr), 8)
    a_off = pl.multiple_of(jnp.minimum(s * ar, n_acc - ar), 8)
    pltpu.sync_copy(msg_hbm.at[pl.ds(t_off, tr)], table_sh.at[pl.ds(t_off, tr)])
    pltpu.sync_copy(zeros_hbm.at[pl.ds(a_off, ar)], acc_sh.at[pl.ds(a_off, ar)])
    plsc.subcore_barrier()

    base = wid * rows_per_tile
    dummy = n_nodes + lax.iota(jnp.int32, DP)

    def chunk(i, carry):
        row0 = base + i * K_CH
        row0c = jnp.minimum(row0, n_rows - K_CH)
        dup = row0 - row0c
        pltpu.sync_copy(edge_hbm.at[0, pl.ds(row0c, K_CH)], src_v)
        pltpu.sync_copy(edge_hbm.at[1, pl.ds(row0c, K_CH)], dst_v)
        for j in range(K_CH):
            @pl.when(j < dup)
            def _():
                for k in range(LANE // DP):
                    dst_v[j, pl.ds(k * DP, DP)] = dummy
        descs = []
        for j in range(K_CH):
            sl = pl.ds(j * LANE, LANE)
            descs.append(
                pltpu.async_copy(table_sh.at[src_v.at[j]], rows_v.at[sl], sem))
        for j in range(K_CH):
            descs[j].wait()
            sl = pl.ds(j * LANE, LANE)
            pltpu.sync_copy(rows_v.at[sl], acc_sh.at[dst_v.at[j]], add=True)
        return carry

    lax.fori_loop(0, rows_per_tile // K_CH, chunk, 0)
    plsc.subcore_barrier()
    pltpu.sync_copy(acc_sh.at[pl.ds(t_off, tr)],
                    out_hbm.at[c, pl.ds(t_off, tr)])


def _make_sc_pass(n_nodes, n_acc, n_rows, rows_per_tile):
    mesh = plsc.VectorSubcoreMesh(core_axis_name="c", subcore_axis_name="s")
    return pl.kernel(
        functools.partial(_sc_scatter, n_nodes, n_acc, n_rows, rows_per_tile),
        out_type=jax.ShapeDtypeStruct((NC, n_nodes, DP), jnp.float32),
        mesh=mesh,
        scratch_types=[
            pltpu.VMEM((K_CH, LANE), jnp.int32),
            pltpu.VMEM((K_CH, LANE), jnp.int32),
            pltpu.VMEM((K_CH * LANE, DP), jnp.float32),
            pltpu.VMEM_SHARED((n_nodes, DP), jnp.float32),
            pltpu.VMEM_SHARED((n_acc, DP), jnp.float32),
            pltpu.SemaphoreType.DMA,
        ],
        compiler_params=pltpu.CompilerParams(use_tc_tiling_on_sc=False),
    )


def _block_diag8(w16):
    return jnp.einsum("ab,kc->akbc", jnp.eye(PK, dtype=jnp.float32),
                      w16).reshape(LANE, LANE)


def kernel(x, edge_index, batch, W1l, W1r, W2l, W2r, Wfc):
    n, f = x.shape
    e = edge_index.shape[1]
    d = W1l.shape[1]
    g = 128
    npk = n // PK
    n_rows = e // LANE
    rows_per_tile = -(-n_rows // (NW * K_CH)) * K_CH
    n_acc = n + DP

    w1l_p = jnp.zeros((f, DP), jnp.float32).at[:, :d].set(W1l)
    w1r_p = jnp.zeros((f, DP), jnp.float32).at[:, :d].set(W1r)
    eye8 = jnp.eye(PK, dtype=jnp.float32)
    w1l_big = jnp.einsum("ab,kc->akbc", eye8, w1l_p).reshape(PK * f, LANE)
    w1r_big = jnp.einsum("ab,kc->akbc", eye8, w1r_p).reshape(PK * f, LANE)
    w2l_p = jnp.zeros((DP, DP), jnp.float32).at[:d, :d].set(W2l).at[d, d].set(1.0)
    w2r_p = jnp.zeros((DP, DP), jnp.float32).at[:d, :d].set(W2r)
    w2l_big = _block_diag8(w2l_p)
    w2r_big = _block_diag8(w2r_p)
    sel = jnp.zeros((DP, DP), jnp.float32).at[d, :].set(1.0)
    s_big = _block_diag8(sel)
    wz_p = jnp.zeros((DP, DP), jnp.float32).at[:d, 0].set(Wfc[:, 0]).at[d, 8].set(1.0)
    wz_big = _block_diag8(wz_p)

    x_pk = x.reshape(npk, PK * f)
    edge3 = edge_index.reshape(2, n_rows, LANE)
    batch_r8 = batch.reshape(npk, PK).T
    zeros = jnp.zeros((n_acc, DP), jnp.float32)

    pre = pl.pallas_call(
        functools.partial(_pre_body, d),
        out_shape=[jax.ShapeDtypeStruct((npk, LANE), jnp.float32)] * 2,
    )
    p1, q1 = pre(x_pk, w1l_big, w1r_big)

    sc_pass = _make_sc_pass(n, n_acc, n_rows, rows_per_tile)
    acc1 = sc_pass(p1.reshape(n, DP), edge3, zeros)

    mid = pl.pallas_call(
        _mid_body,
        out_shape=[jax.ShapeDtypeStruct((npk, LANE), jnp.float32)] * 2,
    )
    p2, q2 = mid(acc1.reshape(NC, npk, LANE), q1, s_big, w2l_big, w2r_big)

    acc2 = sc_pass(p2.reshape(n, DP), edge3, zeros)

    post = pl.pallas_call(
        _post_body,
        out_shape=jax.ShapeDtypeStruct((g, 1), jnp.float32),
    )
    return post(acc2.reshape(NC, npk, LANE), q2, s_big, wz_big, batch_r8)


# speedup vs baseline: 33.6623x; 1.0480x over previous
import functools

import jax
import jax.numpy as jnp
from jax import lax
from jax.experimental import pallas as pl
from jax.experimental.pallas import tpu as pltpu
from jax.experimental.pallas import tpu_sc as plsc

NC = 2
NS = 16
NW = NC * NS
LANE = 128
K_CH = 16
CH = K_CH * LANE
DP = 16
PK = LANE // DP


def _pre_body(one_col, x_ref, wl_ref, wr_ref, p_ref, q_ref):
    wl = wl_ref[...]
    wr = wr_ref[...]
    sel = lax.broadcasted_iota(jnp.int32, (x_ref.shape[0], DP), 1) == one_col
    ps, qs = [], []
    for g in range(PK):
        xg = x_ref[:, g, :]
        pg = jnp.dot(xg, wl, preferred_element_type=jnp.float32)
        ps.append(jnp.where(sel, 1.0, pg))
        qs.append(jnp.dot(xg, wr, preferred_element_type=jnp.float32))
    p_ref[...] = jnp.concatenate(ps, axis=1)
    q_ref[...] = jnp.concatenate(qs, axis=1)


def _mid_body(acc_ref, q_ref, s_ref, wl_ref, wr_ref, p_ref, q2_ref):
    a = acc_ref[0] + acc_ref[1]
    degb = jnp.maximum(jnp.dot(a, s_ref[...],
                               preferred_element_type=jnp.float32), 1.0)
    h = jnp.maximum(a / degb + q_ref[...], 0.0)
    p_ref[...] = jnp.dot(h, wl_ref[...], preferred_element_type=jnp.float32)
    q2_ref[...] = jnp.dot(h, wr_ref[...], preferred_element_type=jnp.float32)


def _post_body(acc_ref, q_ref, s_ref, wz_ref, b_ref, out_ref):
    a = acc_ref[0] + acc_ref[1]
    degb = jnp.maximum(jnp.dot(a, s_ref[...],
                               preferred_element_type=jnp.float32), 1.0)
    h2 = a / degb + q_ref[...]
    z = jnp.dot(h2, wz_ref[...], preferred_element_type=jnp.float32)
    n_graphs = out_ref.shape[0]
    gid = lax.broadcasted_iota(jnp.int32, (n_graphs, b_ref.shape[1]), 0)
    ys = jnp.zeros((n_graphs, 1), jnp.float32)
    cn = jnp.zeros((n_graphs, 1), jnp.float32)
    for j in range(PK):
        oh = (gid == b_ref[j:j + 1, :]).astype(jnp.float32)
        zj = jnp.dot(oh, z, preferred_element_type=jnp.float32)
        ys = ys + zj[:, j * DP:j * DP + 1]
        cn = cn + zj[:, j * DP + 8:j * DP + 9]
    out_ref[...] = jax.nn.sigmoid(ys / jnp.maximum(cn, 1.0))


def _sc_scatter(n_nodes, n_acc, n_edges, ept,
                msg_hbm, edge_hbm, zeros_hbm, out_hbm,
                src_v, dst_v, rows_v, table_sh, acc_sh, sem):
    c = lax.axis_index("c")
    s = lax.axis_index("s")
    wid = s * NC + c
    tr = -(-(n_nodes // NS) // 8) * 8
    ar = -(-(n_acc // NS) // 8) * 8
    t_off = pl.multiple_of(jnp.minimum(s * tr, n_nodes - tr), 8)
    a_off = pl.multiple_of(jnp.minimum(s * ar, n_acc - ar), 8)
    pltpu.sync_copy(msg_hbm.at[pl.ds(t_off, tr)], table_sh.at[pl.ds(t_off, tr)])
    pltpu.sync_copy(zeros_hbm.at[pl.ds(a_off, ar)], acc_sh.at[pl.ds(a_off, ar)])
    plsc.subcore_barrier()

    span0 = wid * ept
    limit = span0 + ept
    dummy = n_nodes + lax.iota(jnp.int32, DP)

    def chunk(i, carry):
        u = span0 + i * CH
        eoff = jnp.minimum(u, n_edges - CH)
        pltpu.sync_copy(edge_hbm.at[0, pl.ds(eoff, CH)], src_v)
        pltpu.sync_copy(edge_hbm.at[1, pl.ds(eoff, CH)], dst_v)
        for j in range(K_CH):
            for k in range(LANE // DP):
                q0 = eoff + j * LANE + k * DP
                @pl.when((q0 < u) | (q0 >= limit))
                def _():
                    dst_v[pl.ds(j * LANE + k * DP, DP)] = dummy
        descs = []
        for j in range(K_CH):
            sl = pl.ds(j * LANE, LANE)
            descs.append(pltpu.async_copy(
                table_sh.at[src_v.at[sl]], rows_v.at[sl], sem))
        for j in range(K_CH):
            descs[j].wait()
            sl = pl.ds(j * LANE, LANE)
            pltpu.sync_copy(rows_v.at[sl], acc_sh.at[dst_v.at[sl]], add=True)
        return carry

    lax.fori_loop(0, -(-ept // CH), chunk, 0)
    plsc.subcore_barrier()
    pltpu.sync_copy(acc_sh.at[pl.ds(t_off, tr)],
                    out_hbm.at[c, pl.ds(t_off, tr)])


def _make_sc_pass(n_nodes, n_acc, n_edges, ept):
    mesh = plsc.VectorSubcoreMesh(core_axis_name="c", subcore_axis_name="s")
    return pl.kernel(
        functools.partial(_sc_scatter, n_nodes, n_acc, n_edges, ept),
        out_type=jax.ShapeDtypeStruct((NC, n_nodes, DP), jnp.float32),
        mesh=mesh,
        scratch_types=[
            pltpu.VMEM((CH,), jnp.int32),
            pltpu.VMEM((CH,), jnp.int32),
            pltpu.VMEM((CH, DP), jnp.float32),
            pltpu.VMEM_SHARED((n_nodes, DP), jnp.float32),
            pltpu.VMEM_SHARED((n_acc, DP), jnp.float32),
            pltpu.SemaphoreType.DMA,
        ],
        compiler_params=pltpu.CompilerParams(use_tc_tiling_on_sc=False),
    )


def _block_diag8(w16):
    return jnp.einsum("ab,kc->akbc", jnp.eye(PK, dtype=jnp.float32),
                      w16).reshape(LANE, LANE)


def kernel(x, edge_index, batch, W1l, W1r, W2l, W2r, Wfc):
    n, f = x.shape
    e = edge_index.shape[1]
    d = W1l.shape[1]
    g = 128
    npk = n // PK
    ept = e // NW
    n_acc = n + DP

    w1l_p = jnp.zeros((f, DP), jnp.float32).at[:, :d].set(W1l)
    w1r_p = jnp.zeros((f, DP), jnp.float32).at[:, :d].set(W1r)
    w2l_p = jnp.zeros((DP, DP), jnp.float32).at[:d, :d].set(W2l).at[d, d].set(1.0)
    w2r_p = jnp.zeros((DP, DP), jnp.float32).at[:d, :d].set(W2r)
    w2l_big = _block_diag8(w2l_p)
    w2r_big = _block_diag8(w2r_p)
    sel = jnp.zeros((DP, DP), jnp.float32).at[d, :].set(1.0)
    s_big = _block_diag8(sel)
    wz_p = jnp.zeros((DP, DP), jnp.float32).at[:d, 0].set(Wfc[:, 0]).at[d, 8].set(1.0)
    wz_big = _block_diag8(wz_p)

    x3 = x.reshape(npk, PK, f)
    batch_r8 = batch.reshape(npk, PK).T
    zeros = jnp.zeros((n_acc, DP), jnp.float32)

    pre = pl.pallas_call(
        functools.partial(_pre_body, d),
        out_shape=[jax.ShapeDtypeStruct((npk, LANE), jnp.float32)] * 2,
    )
    p1, q1 = pre(x3, w1l_p, w1r_p)

    sc_pass = _make_sc_pass(n, n_acc, e, ept)
    acc1 = sc_pass(p1.reshape(n, DP), edge_index, zeros)

    mid = pl.pallas_call(
        _mid_body,
        out_shape=[jax.ShapeDtypeStruct((npk, LANE), jnp.float32)] * 2,
    )
    p2, q2 = mid(acc1.reshape(NC, npk, LANE), q1, s_big, w2l_big, w2r_big)

    acc2 = sc_pass(p2.reshape(n, DP), edge_index, zeros)

    post = pl.pallas_call(
        _post_body,
        out_shape=jax.ShapeDtypeStruct((g, 1), jnp.float32),
    )
    return post(acc2.reshape(NC, npk, LANE), q2, s_big, wz_big, batch_r8)


# speedup vs baseline: 35.2942x; 1.0485x over previous
import functools

import jax
import jax.numpy as jnp
from jax import lax
from jax.experimental import pallas as pl
from jax.experimental.pallas import tpu as pltpu
from jax.experimental.pallas import tpu_sc as plsc

NC = 2
NS = 16
NW = NC * NS
LANE = 128
K_CH = 16
CH = K_CH * LANE
DP = 16
PK = LANE // DP


def _pre_body(one_col, x_ref, wl_ref, wr_ref, p_ref, q_ref):
    wl = wl_ref[...]
    wr = wr_ref[...]
    sel = lax.broadcasted_iota(jnp.int32, (x_ref.shape[0], DP), 1) == one_col
    ps, qs = [], []
    for g in range(PK):
        xg = x_ref[:, g, :]
        pg = jnp.dot(xg, wl, preferred_element_type=jnp.float32)
        ps.append(jnp.where(sel, 1.0, pg))
        qs.append(jnp.dot(xg, wr, preferred_element_type=jnp.float32))
    p_ref[...] = jnp.concatenate(ps, axis=1)
    q_ref[...] = jnp.concatenate(qs, axis=1)


def _mid_body(acc_ref, q_ref, s_ref, wl_ref, wr_ref, p_ref, q2_ref):
    a = acc_ref[0] + acc_ref[1]
    degb = jnp.maximum(jnp.dot(a, s_ref[...],
                               preferred_element_type=jnp.float32), 1.0)
    h = jnp.maximum(a / degb + q_ref[...], 0.0)
    p_ref[...] = jnp.dot(h, wl_ref[...], preferred_element_type=jnp.float32)
    q2_ref[...] = jnp.dot(h, wr_ref[...], preferred_element_type=jnp.float32)


def _post_body(acc_ref, q_ref, s_ref, wz_ref, b_ref, out_ref):
    a = acc_ref[0] + acc_ref[1]
    degb = jnp.maximum(jnp.dot(a, s_ref[...],
                               preferred_element_type=jnp.float32), 1.0)
    h2 = a / degb + q_ref[...]
    z = jnp.dot(h2, wz_ref[...], preferred_element_type=jnp.float32)
    n_graphs = out_ref.shape[0]
    gid = lax.broadcasted_iota(jnp.int32, (n_graphs, b_ref.shape[1]), 0)
    ys = jnp.zeros((n_graphs, 1), jnp.float32)
    cn = jnp.zeros((n_graphs, 1), jnp.float32)
    for j in range(PK):
        oh = (gid == b_ref[j:j + 1, :]).astype(jnp.float32)
        zj = jnp.dot(oh, z, preferred_element_type=jnp.float32)
        ys = ys + zj[:, j * DP:j * DP + 1]
        cn = cn + zj[:, j * DP + 8:j * DP + 9]
    out_ref[...] = jax.nn.sigmoid(ys / jnp.maximum(cn, 1.0))


def _sc_scatter(n_nodes, n_acc, n_edges, ept,
                msg_hbm, edge_hbm, zeros_hbm, out_hbm,
                src_v, dst_v, rows_v, table_sh, acc_sh, sem):
    c = lax.axis_index("c")
    s = lax.axis_index("s")
    wid = s * NC + c
    tr = -(-(n_nodes // NS) // 8) * 8
    ar = -(-(n_acc // NS) // 8) * 8
    t_off = pl.multiple_of(jnp.minimum(s * tr, n_nodes - tr), 8)
    a_off = pl.multiple_of(jnp.minimum(s * ar, n_acc - ar), 8)
    pltpu.sync_copy(msg_hbm.at[pl.ds(t_off, tr)], table_sh.at[pl.ds(t_off, tr)])
    pltpu.sync_copy(zeros_hbm.at[pl.ds(a_off, ar)], acc_sh.at[pl.ds(a_off, ar)])
    plsc.subcore_barrier()

    span0 = wid * ept
    limit = span0 + ept
    dummy = n_nodes + lax.iota(jnp.int32, DP)

    def load_fire(i, b):
        u = span0 + i * CH
        eoff = jnp.minimum(u, n_edges - CH)
        pltpu.sync_copy(edge_hbm.at[0, pl.ds(eoff, CH)], src_v.at[b])
        pltpu.sync_copy(edge_hbm.at[1, pl.ds(eoff, CH)], dst_v.at[b])
        for j in range(K_CH):
            for k in range(LANE // DP):
                q0 = eoff + j * LANE + k * DP
                @pl.when((q0 < u) | (q0 >= limit))
                def _():
                    dst_v[b, pl.ds(j * LANE + k * DP, DP)] = dummy
        descs = []
        for j in range(K_CH):
            sl = pl.ds(j * LANE, LANE)
            descs.append(pltpu.async_copy(
                table_sh.at[src_v.at[b].at[sl]], rows_v.at[b].at[sl], sem))
        return descs

    def drain_scatter(b, descs):
        for j in range(K_CH):
            descs[j].wait()
            sl = pl.ds(j * LANE, LANE)
            pltpu.sync_copy(rows_v.at[b].at[sl],
                            acc_sh.at[dst_v.at[b].at[sl]], add=True)

    n_ch = -(-ept // CH)

    def pair(t, carry):
        i = t * 2
        da = load_fire(i, 0)
        db = load_fire(i + 1, 1)
        drain_scatter(0, da)
        drain_scatter(1, db)
        return carry

    lax.fori_loop(0, n_ch // 2, pair, 0)
    if n_ch % 2:
        drain_scatter(0, load_fire(n_ch - 1, 0))
    plsc.subcore_barrier()
    pltpu.sync_copy(acc_sh.at[pl.ds(t_off, tr)],
                    out_hbm.at[c, pl.ds(t_off, tr)])


def _make_sc_pass(n_nodes, n_acc, n_edges, ept):
    mesh = plsc.VectorSubcoreMesh(core_axis_name="c", subcore_axis_name="s")
    return pl.kernel(
        functools.partial(_sc_scatter, n_nodes, n_acc, n_edges, ept),
        out_type=jax.ShapeDtypeStruct((NC, n_nodes, DP), jnp.float32),
        mesh=mesh,
        scratch_types=[
            pltpu.VMEM((2, CH), jnp.int32),
            pltpu.VMEM((2, CH), jnp.int32),
            pltpu.VMEM((2, CH, DP), jnp.float32),
            pltpu.VMEM_SHARED((n_nodes, DP), jnp.float32),
            pltpu.VMEM_SHARED((n_acc, DP), jnp.float32),
            pltpu.SemaphoreType.DMA,
        ],
        compiler_params=pltpu.CompilerParams(use_tc_tiling_on_sc=False),
    )


def _block_diag8(w16):
    return jnp.einsum("ab,kc->akbc", jnp.eye(PK, dtype=jnp.float32),
                      w16).reshape(LANE, LANE)


def kernel(x, edge_index, batch, W1l, W1r, W2l, W2r, Wfc):
    n, f = x.shape
    e = edge_index.shape[1]
    d = W1l.shape[1]
    g = 128
    npk = n // PK
    ept = e // NW
    n_acc = n + DP

    w1l_p = jnp.zeros((f, DP), jnp.float32).at[:, :d].set(W1l)
    w1r_p = jnp.zeros((f, DP), jnp.float32).at[:, :d].set(W1r)
    w2l_p = jnp.zeros((DP, DP), jnp.float32).at[:d, :d].set(W2l).at[d, d].set(1.0)
    w2r_p = jnp.zeros((DP, DP), jnp.float32).at[:d, :d].set(W2r)
    w2l_big = _block_diag8(w2l_p)
    w2r_big = _block_diag8(w2r_p)
    sel = jnp.zeros((DP, DP), jnp.float32).at[d, :].set(1.0)
    s_big = _block_diag8(sel)
    wz_p = jnp.zeros((DP, DP), jnp.float32).at[:d, 0].set(Wfc[:, 0]).at[d, 8].set(1.0)
    wz_big = _block_diag8(wz_p)

    x3 = x.reshape(npk, PK, f)
    batch_r8 = batch.reshape(npk, PK).T
    zeros = jnp.zeros((n_acc, DP), jnp.float32)

    pre = pl.pallas_call(
        functools.partial(_pre_body, d),
        out_shape=[jax.ShapeDtypeStruct((npk, LANE), jnp.float32)] * 2,
    )
    p1, q1 = pre(x3, w1l_p, w1r_p)

    sc_pass = _make_sc_pass(n, n_acc, e, ept)
    acc1 = sc_pass(p1.reshape(n, DP), edge_index, zeros)

    mid = pl.pallas_call(
        _mid_body,
        out_shape=[jax.ShapeDtypeStruct((npk, LANE), jnp.float32)] * 2,
    )
    p2, q2 = mid(acc1.reshape(NC, npk, LANE), q1, s_big, w2l_big, w2r_big)

    acc2 = sc_pass(p2.reshape(n, DP), edge_index, zeros)

    post = pl.pallas_call(
        _post_body,
        out_shape=jax.ShapeDtypeStruct((g, 1), jnp.float32),
    )
    return post(acc2.reshape(NC, npk, LANE), q2, s_big, wz_big, batch_r8)


# speedup vs baseline: 35.3871x; 1.0026x over previous
import functools

import jax
import jax.numpy as jnp
from jax import lax
from jax.experimental import pallas as pl
from jax.experimental.pallas import tpu as pltpu
from jax.experimental.pallas import tpu_sc as plsc

NC = 2
NS = 16
NW = NC * NS
LANE = 128
K_CH = 16
CH = K_CH * LANE
DP = 16
PK = LANE // DP


def _pre_body(one_col, x_ref, w_ref, p_ref, q_ref):
    w = w_ref[...]
    sel = lax.broadcasted_iota(jnp.int32, (x_ref.shape[0], DP), 1) == one_col
    ps, qs = [], []
    for g in range(PK):
        pq = jnp.dot(x_ref[:, g, :], w, preferred_element_type=jnp.float32)
        ps.append(jnp.where(sel, 1.0, pq[:, :DP]))
        qs.append(pq[:, DP:])
    p_ref[...] = jnp.concatenate(ps, axis=1)
    q_ref[...] = jnp.concatenate(qs, axis=1)


def _mid_body(acc_ref, q_ref, s_ref, wl_ref, wr_ref, p_ref, q2_ref):
    a = acc_ref[0] + acc_ref[1]
    degb = jnp.maximum(jnp.dot(a, s_ref[...],
                               preferred_element_type=jnp.float32), 1.0)
    h = jnp.maximum(a / degb + q_ref[...], 0.0)
    p_ref[...] = jnp.dot(h, wl_ref[...], preferred_element_type=jnp.float32)
    q2_ref[...] = jnp.dot(h, wr_ref[...], preferred_element_type=jnp.float32)


def _post_body(acc_ref, q_ref, s_ref, wz_ref, b_ref, out_ref):
    a = acc_ref[0] + acc_ref[1]
    degb = jnp.maximum(jnp.dot(a, s_ref[...],
                               preferred_element_type=jnp.float32), 1.0)
    h2 = a / degb + q_ref[...]
    z = jnp.dot(h2, wz_ref[...], preferred_element_type=jnp.float32)
    n_graphs = out_ref.shape[0]
    npk = b_ref.shape[0]
    gid = lax.broadcasted_iota(jnp.int32, (npk, n_graphs), 1)
    ys = jnp.zeros((n_graphs, 1), jnp.float32)
    cn = jnp.zeros((n_graphs, 1), jnp.float32)
    for j in range(PK):
        oht = (b_ref[:, j:j + 1] == gid).astype(jnp.float32)
        zj = lax.dot_general(oht, z, (((0,), (0,)), ((), ())),
                             preferred_element_type=jnp.float32)
        ys = ys + zj[:, j * DP:j * DP + 1]
        cn = cn + zj[:, j * DP + 8:j * DP + 9]
    out_ref[...] = jax.nn.sigmoid(ys / jnp.maximum(cn, 1.0))


def _sc_scatter(n_nodes, n_acc, n_rows,
                msg_hbm, edge_hbm, zeros_hbm, out_hbm,
                src_v, dst_v, rows_v, table_sh, acc_sh, sem):
    c = lax.axis_index("c")
    s = lax.axis_index("s")
    wid = s * NC + c
    tr = -(-(n_nodes // NS) // 8) * 8
    ar = -(-(n_acc // NS) // 8) * 8
    t_off = pl.multiple_of(jnp.minimum(s * tr, n_nodes - tr), 8)
    a_off = pl.multiple_of(jnp.minimum(s * ar, n_acc - ar), 8)
    pltpu.sync_copy(msg_hbm.at[pl.ds(t_off, tr)], table_sh.at[pl.ds(t_off, tr)])
    pltpu.sync_copy(zeros_hbm.at[pl.ds(a_off, ar)], acc_sh.at[pl.ds(a_off, ar)])
    plsc.subcore_barrier()

    rs0 = (wid * n_rows) // NW
    rs1 = ((wid + 1) * n_rows) // NW
    dummy = n_nodes + lax.iota(jnp.int32, DP)

    def load_fire(i, b):
        row0 = jnp.minimum(rs0 + i * K_CH, n_rows - K_CH)
        pltpu.sync_copy(edge_hbm.at[pl.ds(row0, K_CH), 0], src_v.at[b])
        pltpu.sync_copy(edge_hbm.at[pl.ds(row0, K_CH), 1], dst_v.at[b])
        for j in range(K_CH):
            @pl.when((row0 + j < rs0) | (row0 + j >= rs1))
            def _():
                for k in range(LANE // DP):
                    dst_v[b, j, pl.ds(k * DP, DP)] = dummy
        descs = []
        for j in range(K_CH):
            descs.append(pltpu.async_copy(
                table_sh.at[src_v.at[b].at[j]],
                rows_v.at[b].at[pl.ds(j * LANE, LANE)], sem))
        return descs

    def drain_scatter(b, descs):
        for j in range(K_CH):
            descs[j].wait()
            pltpu.sync_copy(rows_v.at[b].at[pl.ds(j * LANE, LANE)],
                            acc_sh.at[dst_v.at[b].at[j]], add=True)

    n_ch = -(--(-n_rows // NW) // K_CH)

    def pair(t, carry):
        i = t * 2
        da = load_fire(i, 0)
        db = load_fire(i + 1, 1)
        drain_scatter(0, da)
        drain_scatter(1, db)
        return carry

    lax.fori_loop(0, n_ch // 2, pair, 0)
    if n_ch % 2:
        drain_scatter(0, load_fire(n_ch - 1, 0))
    plsc.subcore_barrier()
    pltpu.sync_copy(acc_sh.at[pl.ds(t_off, tr)],
                    out_hbm.at[c, pl.ds(t_off, tr)])


def _make_sc_pass(n_nodes, n_acc, n_rows):
    mesh = plsc.VectorSubcoreMesh(core_axis_name="c", subcore_axis_name="s")
    return pl.kernel(
        functools.partial(_sc_scatter, n_nodes, n_acc, n_rows),
        out_type=jax.ShapeDtypeStruct((NC, n_nodes, DP), jnp.float32),
        mesh=mesh,
        scratch_types=[
            pltpu.VMEM((2, K_CH, LANE), jnp.int32),
            pltpu.VMEM((2, K_CH, LANE), jnp.int32),
            pltpu.VMEM((2, CH, DP), jnp.float32),
            pltpu.VMEM_SHARED((n_nodes, DP), jnp.float32),
            pltpu.VMEM_SHARED((n_acc, DP), jnp.float32),
            pltpu.SemaphoreType.DMA,
        ],
        compiler_params=pltpu.CompilerParams(use_tc_tiling_on_sc=False),
    )


def _block_diag8(w16):
    return jnp.einsum("ab,kc->akbc", jnp.eye(PK, dtype=jnp.float32),
                      w16).reshape(LANE, LANE)


def kernel(x, edge_index, batch, W1l, W1r, W2l, W2r, Wfc):
    n, f = x.shape
    e = edge_index.shape[1]
    d = W1l.shape[1]
    g = 128
    npk = n // PK
    n_rows = e // LANE
    n_acc = n + DP

    w1_p = (jnp.zeros((f, 2 * DP), jnp.float32)
            .at[:, :d].set(W1l).at[:, DP:DP + d].set(W1r))
    w2l_p = jnp.zeros((DP, DP), jnp.float32).at[:d, :d].set(W2l).at[d, d].set(1.0)
    w2r_p = jnp.zeros((DP, DP), jnp.float32).at[:d, :d].set(W2r)
    w2l_big = _block_diag8(w2l_p)
    w2r_big = _block_diag8(w2r_p)
    sel = jnp.zeros((DP, DP), jnp.float32).at[d, :].set(1.0)
    s_big = _block_diag8(sel)
    wz_p = jnp.zeros((DP, DP), jnp.float32).at[:d, 0].set(Wfc[:, 0]).at[d, 8].set(1.0)
    wz_big = _block_diag8(wz_p)

    x3 = x.reshape(npk, PK, f)
    edge3i = edge_index.reshape(2, n_rows, LANE).transpose(1, 0, 2)
    batch_p = batch.reshape(npk, PK)
    zeros = jnp.zeros((n_acc, DP), jnp.float32)

    pre = pl.pallas_call(
        functools.partial(_pre_body, d),
        out_shape=[jax.ShapeDtypeStruct((npk, LANE), jnp.float32)] * 2,
    )
    p1, q1 = pre(x3, w1_p)

    sc_pass = _make_sc_pass(n, n_acc, n_rows)
    acc1 = sc_pass(p1.reshape(n, DP), edge3i, zeros)

    mid = pl.pallas_call(
        _mid_body,
        out_shape=[jax.ShapeDtypeStruct((npk, LANE), jnp.float32)] * 2,
    )
    p2, q2 = mid(acc1.reshape(NC, npk, LANE), q1, s_big, w2l_big, w2r_big)

    acc2 = sc_pass(p2.reshape(n, DP), edge3i, zeros)

    post = pl.pallas_call(
        _post_body,
        out_shape=jax.ShapeDtypeStruct((g, 1), jnp.float32),
    )
    return post(acc2.reshape(NC, npk, LANE), q2, s_big, wz_big, batch_p)


# speedup vs baseline: 35.7624x; 1.0106x over previous
import functools

import jax
import jax.numpy as jnp
from jax import lax
from jax.experimental import pallas as pl
from jax.experimental.pallas import tpu as pltpu
from jax.experimental.pallas import tpu_sc as plsc

NC = 2
NS = 16
NW = NC * NS
LANE = 128
K_CH = 16
CH = K_CH * LANE
DP = 16
PK = LANE // DP


def _pre_body(one_col, x_ref, w_ref, p_ref, q_ref):
    w = w_ref[...]
    sel = lax.broadcasted_iota(jnp.int32, (x_ref.shape[0], DP), 1) == one_col
    ps, qs = [], []
    for g in range(PK):
        pq = jnp.dot(x_ref[:, g, :], w, preferred_element_type=jnp.float32)
        ps.append(jnp.where(sel, 1.0, pq[:, :DP]))
        qs.append(pq[:, DP:])
    p_ref[...] = jnp.concatenate(ps, axis=1)
    q_ref[...] = jnp.concatenate(qs, axis=1)


def _mid_body(acc_ref, q_ref, s_ref, wl_ref, wr_ref, p_ref, q2_ref):
    a = acc_ref[0] + acc_ref[1]
    degb = jnp.maximum(jnp.dot(a, s_ref[...],
                               preferred_element_type=jnp.float32), 1.0)
    h = jnp.maximum(a / degb + q_ref[...], 0.0)
    p_ref[...] = jnp.dot(h, wl_ref[...], preferred_element_type=jnp.float32)
    q2_ref[...] = jnp.dot(h, wr_ref[...], preferred_element_type=jnp.float32)


def _post_body(acc_ref, q_ref, s_ref, wz_ref, b_ref, out_ref):
    a = acc_ref[0] + acc_ref[1]
    degb = jnp.maximum(jnp.dot(a, s_ref[...],
                               preferred_element_type=jnp.float32), 1.0)
    h2 = a / degb + q_ref[...]
    z = jnp.dot(h2, wz_ref[...], preferred_element_type=jnp.float32)
    n_graphs = out_ref.shape[0]
    gid = lax.broadcasted_iota(jnp.int32, (n_graphs, b_ref.shape[1]), 0)
    ys = jnp.zeros((n_graphs, 1), jnp.float32)
    cn = jnp.zeros((n_graphs, 1), jnp.float32)
    for j in range(PK):
        oh = (gid == b_ref[j:j + 1, :]).astype(jnp.float32)
        zj = jnp.dot(oh, z, preferred_element_type=jnp.float32)
        ys = ys + zj[:, j * DP:j * DP + 1]
        cn = cn + zj[:, j * DP + 8:j * DP + 9]
    out_ref[...] = jax.nn.sigmoid(ys / jnp.maximum(cn, 1.0))


def _sc_scatter(n_nodes, n_acc, n_rows,
                msg_hbm, edge_hbm, zeros_hbm, out_hbm,
                src_v, dst_v, rows_v, table_sh, acc_sh, sem):
    c = lax.axis_index("c")
    s = lax.axis_index("s")
    wid = s * NC + c
    tr = -(-(n_nodes // NS) // 8) * 8
    ar = -(-(n_acc // NS) // 8) * 8
    t_off = pl.multiple_of(jnp.minimum(s * tr, n_nodes - tr), 8)
    a_off = pl.multiple_of(jnp.minimum(s * ar, n_acc - ar), 8)
    pltpu.sync_copy(msg_hbm.at[pl.ds(t_off, tr)], table_sh.at[pl.ds(t_off, tr)])
    pltpu.sync_copy(zeros_hbm.at[pl.ds(a_off, ar)], acc_sh.at[pl.ds(a_off, ar)])
    plsc.subcore_barrier()

    rs0 = (wid * n_rows) // NW
    rs1 = ((wid + 1) * n_rows) // NW
    dummy = n_nodes + lax.iota(jnp.int32, DP)

    def load_fire(i, b):
        row0 = jnp.minimum(rs0 + i * K_CH, n_rows - K_CH)
        pltpu.sync_copy(edge_hbm.at[pl.ds(row0, K_CH), 0], src_v.at[b])
        pltpu.sync_copy(edge_hbm.at[pl.ds(row0, K_CH), 1], dst_v.at[b])
        for j in range(K_CH):
            @pl.when((row0 + j < rs0) | (row0 + j >= rs1))
            def _():
                for k in range(LANE // DP):
                    dst_v[b, j, pl.ds(k * DP, DP)] = dummy
        descs = []
        for j in range(K_CH):
            descs.append(pltpu.async_copy(
                table_sh.at[src_v.at[b].at[j]],
                rows_v.at[b].at[pl.ds(j * LANE, LANE)], sem))
        return descs

    def drain_scatter(b, descs):
        for j in range(K_CH):
            descs[j].wait()
            pltpu.sync_copy(rows_v.at[b].at[pl.ds(j * LANE, LANE)],
                            acc_sh.at[dst_v.at[b].at[j]], add=True)

    n_ch = -(--(-n_rows // NW) // K_CH)

    def pair(t, carry):
        i = t * 2
        da = load_fire(i, 0)
        db = load_fire(i + 1, 1)
        drain_scatter(0, da)
        drain_scatter(1, db)
        return carry

    lax.fori_loop(0, n_ch // 2, pair, 0)
    if n_ch % 2:
        drain_scatter(0, load_fire(n_ch - 1, 0))
    plsc.subcore_barrier()
    pltpu.sync_copy(acc_sh.at[pl.ds(t_off, tr)],
                    out_hbm.at[c, pl.ds(t_off, tr)])


def _make_sc_pass(n_nodes, n_acc, n_rows):
    mesh = plsc.VectorSubcoreMesh(core_axis_name="c", subcore_axis_name="s")
    return pl.kernel(
        functools.partial(_sc_scatter, n_nodes, n_acc, n_rows),
        out_type=jax.ShapeDtypeStruct((NC, n_nodes, DP), jnp.float32),
        mesh=mesh,
        scratch_types=[
            pltpu.VMEM((2, K_CH, LANE), jnp.int32),
            pltpu.VMEM((2, K_CH, LANE), jnp.int32),
            pltpu.VMEM((2, CH, DP), jnp.float32),
            pltpu.VMEM_SHARED((n_nodes, DP), jnp.float32),
            pltpu.VMEM_SHARED((n_acc, DP), jnp.float32),
            pltpu.SemaphoreType.DMA,
        ],
        compiler_params=pltpu.CompilerParams(use_tc_tiling_on_sc=False),
    )


def _block_diag8(w16):
    return jnp.einsum("ab,kc->akbc", jnp.eye(PK, dtype=jnp.float32),
                      w16).reshape(LANE, LANE)


def kernel(x, edge_index, batch, W1l, W1r, W2l, W2r, Wfc):
    n, f = x.shape
    e = edge_index.shape[1]
    d = W1l.shape[1]
    g = 128
    npk = n // PK
    n_rows = e // LANE
    n_acc = n + DP

    w1_p = (jnp.zeros((f, 2 * DP), jnp.float32)
            .at[:, :d].set(W1l).at[:, DP:DP + d].set(W1r))
    w2l_p = jnp.zeros((DP, DP), jnp.float32).at[:d, :d].set(W2l).at[d, d].set(1.0)
    w2r_p = jnp.zeros((DP, DP), jnp.float32).at[:d, :d].set(W2r)
    w2l_big = _block_diag8(w2l_p)
    w2r_big = _block_diag8(w2r_p)
    sel = jnp.zeros((DP, DP), jnp.float32).at[d, :].set(1.0)
    s_big = _block_diag8(sel)
    wz_p = jnp.zeros((DP, DP), jnp.float32).at[:d, 0].set(Wfc[:, 0]).at[d, 8].set(1.0)
    wz_big = _block_diag8(wz_p)

    x3 = x.reshape(npk, PK, f)
    edge3i = edge_index.reshape(2, n_rows, LANE).transpose(1, 0, 2)
    batch_r8 = batch.reshape(npk, PK).T
    zeros = jnp.zeros((n_acc, DP), jnp.float32)

    rb = 128
    nblk = -(-npk // rb)
    pre = pl.pallas_call(
        functools.partial(_pre_body, d),
        grid=(nblk,),
        in_specs=[
            pl.BlockSpec((rb, PK, f), lambda i: (i, 0, 0)),
            pl.BlockSpec((f, 2 * DP), lambda i: (0, 0)),
        ],
        out_specs=[pl.BlockSpec((rb, LANE), lambda i: (i, 0))] * 2,
        out_shape=[jax.ShapeDtypeStruct((npk, LANE), jnp.float32)] * 2,
    )
    p1, q1 = pre(x3, w1_p)

    sc_pass = _make_sc_pass(n, n_acc, n_rows)
    acc1 = sc_pass(p1.reshape(n, DP), edge3i, zeros)

    mid = pl.pallas_call(
        _mid_body,
        out_shape=[jax.ShapeDtypeStruct((npk, LANE), jnp.float32)] * 2,
    )
    p2, q2 = mid(acc1.reshape(NC, npk, LANE), q1, s_big, w2l_big, w2r_big)

    acc2 = sc_pass(p2.reshape(n, DP), edge3i, zeros)

    post = pl.pallas_call(
        _post_body,
        out_shape=jax.ShapeDtypeStruct((g, 1), jnp.float32),
    )
    return post(acc2.reshape(NC, npk, LANE), q2, s_big, wz_big, batch_r8)


# speedup vs baseline: 36.6610x; 1.0251x over previous
import functools

import jax
import jax.numpy as jnp
from jax import lax
from jax.experimental import pallas as pl
from jax.experimental.pallas import tpu as pltpu
from jax.experimental.pallas import tpu_sc as plsc

NC = 2
NS = 16
NW = NC * NS
LANE = 128
K_CH = 16
CH = K_CH * LANE
DP = 16
PK = LANE // DP


def _pre_body(one_col, x_ref, wl_ref, wr_ref, p_ref, q_ref):
    f = wl_ref.shape[0]
    d = wl_ref.shape[1]
    zpad = jnp.zeros((f, DP - d), jnp.float32)
    w = jnp.concatenate([wl_ref[...], zpad, wr_ref[...], zpad], axis=1)
    sel = lax.broadcasted_iota(jnp.int32, (x_ref.shape[0], DP), 1) == one_col
    ps, qs = [], []
    for g in range(PK):
        pq = jnp.dot(x_ref[:, g, :], w, preferred_element_type=jnp.float32)
        ps.append(jnp.where(sel, 1.0, pq[:, :DP]))
        qs.append(pq[:, DP:])
    p_ref[...] = jnp.concatenate(ps, axis=1)
    q_ref[...] = jnp.concatenate(qs, axis=1)


def _mid_body(acc_ref, q_ref, s_ref, wl_ref, wr_ref, p_ref, q2_ref):
    a = acc_ref[0] + acc_ref[1]
    degb = jnp.maximum(jnp.dot(a, s_ref[...],
                               preferred_element_type=jnp.float32), 1.0)
    h = jnp.maximum(a / degb + q_ref[...], 0.0)
    p_ref[...] = jnp.dot(h, wl_ref[...], preferred_element_type=jnp.float32)
    q2_ref[...] = jnp.dot(h, wr_ref[...], preferred_element_type=jnp.float32)


def _post_body(acc_ref, q_ref, s_ref, wz_ref, b_ref, out_ref):
    a = acc_ref[0] + acc_ref[1]
    degb = jnp.maximum(jnp.dot(a, s_ref[...],
                               preferred_element_type=jnp.float32), 1.0)
    h2 = a / degb + q_ref[...]
    z = jnp.dot(h2, wz_ref[...], preferred_element_type=jnp.float32)
    n_graphs = out_ref.shape[0]
    gid = lax.broadcasted_iota(jnp.int32, (n_graphs, b_ref.shape[1]), 0)
    ys = jnp.zeros((n_graphs, 1), jnp.float32)
    cn = jnp.zeros((n_graphs, 1), jnp.float32)
    for j in range(PK):
        oh = (gid == b_ref[j:j + 1, :]).astype(jnp.float32)
        zj = jnp.dot(oh, z, preferred_element_type=jnp.float32)
        ys = ys + zj[:, j * DP:j * DP + 1]
        cn = cn + zj[:, j * DP + 8:j * DP + 9]
    out_ref[...] = jax.nn.sigmoid(ys / jnp.maximum(cn, 1.0))


def _sc_scatter(n_nodes, n_acc, n_rows,
                msg_hbm, edge_hbm, zeros_hbm, out_hbm,
                src_v, dst_v, rows_v, table_sh, acc_sh, sem):
    c = lax.axis_index("c")
    s = lax.axis_index("s")
    wid = s * NC + c
    tr = -(-(n_nodes // NS) // 8) * 8
    ar = -(-(n_acc // NS) // 8) * 8
    t_off = pl.multiple_of(jnp.minimum(s * tr, n_nodes - tr), 8)
    a_off = pl.multiple_of(jnp.minimum(s * ar, n_acc - ar), 8)
    pltpu.sync_copy(msg_hbm.at[pl.ds(t_off, tr)], table_sh.at[pl.ds(t_off, tr)])
    pltpu.sync_copy(zeros_hbm.at[pl.ds(a_off, ar)], acc_sh.at[pl.ds(a_off, ar)])
    plsc.subcore_barrier()

    rs0 = (wid * n_rows) // NW
    rs1 = ((wid + 1) * n_rows) // NW
    dummy = n_nodes + lax.iota(jnp.int32, DP)

    def load_fire(i, b):
        row0 = jnp.minimum(rs0 + i * K_CH, n_rows - K_CH)
        pltpu.sync_copy(edge_hbm.at[pl.ds(row0, K_CH), 0], src_v.at[b])
        pltpu.sync_copy(edge_hbm.at[pl.ds(row0, K_CH), 1], dst_v.at[b])
        for j in range(K_CH):
            @pl.when((row0 + j < rs0) | (row0 + j >= rs1))
            def _():
                for k in range(LANE // DP):
                    dst_v[b, j, pl.ds(k * DP, DP)] = dummy
        descs = []
        for j in range(K_CH):
            descs.append(pltpu.async_copy(
                table_sh.at[src_v.at[b].at[j]],
                rows_v.at[b].at[pl.ds(j * LANE, LANE)], sem))
        return descs

    def drain_scatter(b, descs):
        for j in range(K_CH):
            descs[j].wait()
            pltpu.sync_copy(rows_v.at[b].at[pl.ds(j * LANE, LANE)],
                            acc_sh.at[dst_v.at[b].at[j]], add=True)

    n_ch = -(--(-n_rows // NW) // K_CH)

    def pair(t, carry):
        i = t * 2
        da = load_fire(i, 0)
        db = load_fire(i + 1, 1)
        drain_scatter(0, da)
        drain_scatter(1, db)
        return carry

    lax.fori_loop(0, n_ch // 2, pair, 0)
    if n_ch % 2:
        drain_scatter(0, load_fire(n_ch - 1, 0))
    plsc.subcore_barrier()
    pltpu.sync_copy(acc_sh.at[pl.ds(t_off, tr)],
                    out_hbm.at[c, pl.ds(t_off, tr)])


def _make_sc_pass(n_nodes, n_acc, n_rows):
    mesh = plsc.VectorSubcoreMesh(core_axis_name="c", subcore_axis_name="s")
    return pl.kernel(
        functools.partial(_sc_scatter, n_nodes, n_acc, n_rows),
        out_type=jax.ShapeDtypeStruct((NC, n_nodes, DP), jnp.float32),
        mesh=mesh,
        scratch_types=[
            pltpu.VMEM((2, K_CH, LANE), jnp.int32),
            pltpu.VMEM((2, K_CH, LANE), jnp.int32),
            pltpu.VMEM((2, CH, DP), jnp.float32),
            pltpu.VMEM_SHARED((n_nodes, DP), jnp.float32),
            pltpu.VMEM_SHARED((n_acc, DP), jnp.float32),
            pltpu.SemaphoreType.DMA,
        ],
        compiler_params=pltpu.CompilerParams(use_tc_tiling_on_sc=False),
    )


def _block_diag8(w16):
    return jnp.einsum("ab,kc->akbc", jnp.eye(PK, dtype=jnp.float32),
                      w16).reshape(LANE, LANE)


def kernel(x, edge_index, batch, W1l, W1r, W2l, W2r, Wfc):
    n, f = x.shape
    e = edge_index.shape[1]
    d = W1l.shape[1]
    g = 128
    npk = n // PK
    n_rows = e // LANE
    n_acc = n + DP


    w2l_p = jnp.zeros((DP, DP), jnp.float32).at[:d, :d].set(W2l).at[d, d].set(1.0)
    w2r_p = jnp.zeros((DP, DP), jnp.float32).at[:d, :d].set(W2r)
    w2l_big = _block_diag8(w2l_p)
    w2r_big = _block_diag8(w2r_p)
    sel = jnp.zeros((DP, DP), jnp.float32).at[d, :].set(1.0)
    s_big = _block_diag8(sel)
    wz_p = jnp.zeros((DP, DP), jnp.float32).at[:d, 0].set(Wfc[:, 0]).at[d, 8].set(1.0)
    wz_big = _block_diag8(wz_p)

    x3 = x.reshape(npk, PK, f)
    edge3i = edge_index.reshape(2, n_rows, LANE).transpose(1, 0, 2)
    batch_r8 = batch.reshape(npk, PK).T
    zeros = jnp.zeros((n_acc, DP), jnp.float32)

    rb = 128
    nblk = -(-npk // rb)
    pre = pl.pallas_call(
        functools.partial(_pre_body, d),
        grid=(nblk,),
        in_specs=[
            pl.BlockSpec((rb, PK, f), lambda i: (i, 0, 0)),
            pl.BlockSpec((f, d), lambda i: (0, 0)),
            pl.BlockSpec((f, d), lambda i: (0, 0)),
        ],
        out_specs=[pl.BlockSpec((rb, LANE), lambda i: (i, 0))] * 2,
        out_shape=[jax.ShapeDtypeStruct((npk, LANE), jnp.float32)] * 2,
    )
    p1, q1 = pre(x3, W1l, W1r)

    sc_pass = _make_sc_pass(n, n_acc, n_rows)
    acc1 = sc_pass(p1.reshape(n, DP), edge3i, zeros)

    mid = pl.pallas_call(
        _mid_body,
        out_shape=[jax.ShapeDtypeStruct((npk, LANE), jnp.float32)] * 2,
    )
    p2, q2 = mid(acc1.reshape(NC, npk, LANE), q1, s_big, w2l_big, w2r_big)

    acc2 = sc_pass(p2.reshape(n, DP), edge3i, zeros)

    post = pl.pallas_call(
        _post_body,
        out_shape=jax.ShapeDtypeStruct((g, 1), jnp.float32),
    )
    return post(acc2.reshape(NC, npk, LANE), q2, s_big, wz_big, batch_r8)
